# baseline (speedup 1.0000x reference)
import functools

import jax
import jax.numpy as jnp
from jax import lax
from jax.experimental import pallas as pl
from jax.experimental.pallas import tpu as pltpu
from jax.experimental.pallas import tpu_sc as plsc

_NUM_EMBEDDINGS = 100000
_EM_SIZE = 64
_PAD = 128
_BATCH = 16384
_HALF = 50048

_info = plsc.get_sparse_core_info()
_NC, _NS = _info.num_cores, _info.num_subcores
_NW = _NC * _NS
_B_PER_W = _BATCH // _NW

_mesh = plsc.VectorSubcoreMesh(core_axis_name="c", subcore_axis_name="s")

_TBLK = 2944
_TGRID = _HALF // _TBLK


def _transpose_body(lo_ref, hi_ref, o_ref):
    o_ref[:, : _EM_SIZE] = jnp.transpose(lo_ref[...], (1, 0))
    o_ref[:, _EM_SIZE :] = jnp.transpose(hi_ref[...], (1, 0))


_transpose_call = pl.pallas_call(
    _transpose_body,
    grid=(_TGRID,),
    in_specs=[
        pl.BlockSpec((_EM_SIZE, _TBLK), lambda i: (0, i)),
        pl.BlockSpec((_EM_SIZE, _TBLK), lambda i: (0, i + _TGRID)),
    ],
    out_specs=pl.BlockSpec((_TBLK, _PAD), lambda i: (i, 0)),
    out_shape=jax.ShapeDtypeStruct((_HALF, _PAD), jnp.float32),
)


@functools.partial(
    pl.kernel,
    mesh=_mesh,
    out_type=jax.ShapeDtypeStruct((_BATCH, _PAD), jnp.float32),
    scratch_types=[
        pltpu.VMEM((_B_PER_W,), jnp.int32),
        pltpu.VMEM((_B_PER_W,), jnp.int32),
        pltpu.VMEM((_B_PER_W,), jnp.int32),
        pltpu.VMEM((_B_PER_W, _PAD), jnp.float32),
        pltpu.VMEM((_B_PER_W, _EM_SIZE), jnp.float32),
        pltpu.SemaphoreType.DMA,
    ],
    compiler_params=pltpu.CompilerParams(
        use_tc_tiling_on_sc=False,
        disable_bounds_checks=True,
        disable_semaphore_checks=True,
    ),
)
def _gather_kernel(
    idx_hbm, table_hbm, out_hbm, idx_v, idx2_v, off_v, rows_v, out_v, sem
):
    wid = lax.axis_index("s") * _NC + lax.axis_index("c")
    base = wid * _B_PER_W
    pltpu.sync_copy(idx_hbm.at[pl.ds(base, _B_PER_W)], idx_v)

    def fold(i, _):
        v = idx_v[pl.ds(i * 16, 16)]
        idx2_v[pl.ds(i * 16, 16)] = lax.rem(v, _HALF)
        off_v[pl.ds(i * 16, 16)] = lax.div(v, _HALF) * _EM_SIZE
        return _

    lax.fori_loop(0, _B_PER_W // 16, fold, 0)
    pltpu.async_copy(table_hbm.at[idx2_v], rows_v, sem).wait()

    def compact(i, _):
        ov = off_v[pl.ds(i * 16, 16)]
        for j in range(16):
            b = i * 16 + j
            off = ov[j]
            for k in range(_EM_SIZE // 16):
                out_v[b, pl.ds(k * 16, 16)] = rows_v[b, pl.ds(off + k * 16, 16)]
        return _

    lax.fori_loop(0, _B_PER_W // 16, compact, 0)
    pltpu.sync_copy(
        out_v, out_hbm.at[pl.ds(base, _B_PER_W), pl.ds(0, _EM_SIZE)]
    )


def kernel(hyperparameters, table):
    idx = jnp.reshape(hyperparameters, (_BATCH,)).astype(jnp.int32)
    table_stag = _transpose_call(table.T, table.T)
    out_pad = _gather_kernel(idx, table_stag)
    return out_pad[:, :_EM_SIZE]


# speedup vs baseline: 1.1898x; 1.1898x over previous
import functools

import jax
import jax.numpy as jnp
from jax import lax
from jax.experimental import pallas as pl
from jax.experimental.pallas import tpu as pltpu
from jax.experimental.pallas import tpu_sc as plsc

_NUM_EMBEDDINGS = 100000
_EM_SIZE = 64
_PAD = 128
_BATCH = 16384

_info = plsc.get_sparse_core_info()
_NC, _NS = _info.num_cores, _info.num_subcores
_NW = _NC * _NS
_B_PER_W = _BATCH // _NW

_mesh = plsc.VectorSubcoreMesh(core_axis_name="c", subcore_axis_name="s")

_TBLK = 32768
_TGRID = -(-_NUM_EMBEDDINGS // _TBLK)


def _transpose_body(t_ref, o_ref):
    for c in range(_TBLK // 1024):
        o_ref[pl.ds(c * 1024, 1024), : _EM_SIZE] = jnp.transpose(
            t_ref[:, pl.ds(c * 1024, 1024)], (1, 0)
        )


_transpose_call = pl.pallas_call(
    _transpose_body,
    grid=(_TGRID,),
    in_specs=[pl.BlockSpec((_EM_SIZE, _TBLK), lambda i: (0, i))],
    out_specs=pl.BlockSpec((_TBLK, _PAD), lambda i: (i, 0)),
    out_shape=jax.ShapeDtypeStruct((_NUM_EMBEDDINGS, _PAD), jnp.float32),
)


@functools.partial(
    pl.kernel,
    mesh=_mesh,
    out_type=jax.ShapeDtypeStruct((_BATCH, _PAD), jnp.float32),
    scratch_types=[
        pltpu.VMEM((_B_PER_W,), jnp.int32),
        pltpu.VMEM((_B_PER_W, _PAD), jnp.float32),
        pltpu.SemaphoreType.DMA,
    ],
    compiler_params=pltpu.CompilerParams(
        use_tc_tiling_on_sc=False,
        disable_bounds_checks=True,
        disable_semaphore_checks=True,
        skip_device_barrier=True,
    ),
)
def _gather_kernel(idx_hbm, table_hbm, out_hbm, idx_v, rows_v, sem):
    wid = lax.axis_index("s") * _NC + lax.axis_index("c")
    base = wid * _B_PER_W
    pltpu.sync_copy(idx_hbm.at[pl.ds(base, _B_PER_W)], idx_v)
    pltpu.async_copy(table_hbm.at[idx_v], rows_v, sem).wait()
    pltpu.sync_copy(rows_v, out_hbm.at[pl.ds(base, _B_PER_W)])


def kernel(hyperparameters, table):
    idx = jnp.reshape(hyperparameters, (_BATCH,)).astype(jnp.int32)
    table_pad = _transpose_call(table.T)
    out_pad = _gather_kernel(idx, table_pad)
    return out_pad[:, :_EM_SIZE]
